# baseline (speedup 1.0000x reference)
import functools

import jax
import jax.numpy as jnp
from jax import lax
from jax.experimental import pallas as pl
from jax.experimental.pallas import tpu as pltpu
from jax.experimental.pallas import tpu_sc as plsc

_NUM_CORES = 2
_NUM_SUBCORES = 16
_NW = _NUM_CORES * _NUM_SUBCORES

_D = 128
_CHUNK = 80
_NB = 3


@functools.partial(jax.jit, static_argnames=("n_edges",))
def _sc_gather(x, src, *, n_edges):
    per_w = n_edges // _NW
    n_full = per_w // _CHUNK
    tail_c = per_w - n_full * _CHUNK
    n_rounds = n_full // _NB
    assert n_full >= _NB and tail_c % 8 == 0
    n_nodes = x.shape[0]
    rows_per_s = n_nodes // _NUM_SUBCORES
    mesh = plsc.VectorSubcoreMesh(core_axis_name="c", subcore_axis_name="s")

    scratch = (
        [
            pltpu.VMEM_SHARED((n_nodes, _D), jnp.float32),
            pltpu.VMEM((per_w,), jnp.int32),
        ]
        + [pltpu.VMEM((_CHUNK, _D), jnp.float32) for _ in range(_NB)]
        + [pltpu.SemaphoreType.DMA for _ in range(2 * _NB)]
    )

    @functools.partial(
        pl.kernel,
        out_type=jax.ShapeDtypeStruct((n_edges, _D), jnp.float32),
        mesh=mesh,
        scratch_types=scratch,
    )
    def body(x_hbm, src_hbm, out_hbm, xs, idx_v, *bufs_and_sems):
        bufs = bufs_and_sems[:_NB]
        gsem = bufs_and_sems[_NB : 2 * _NB]
        wsem = bufs_and_sems[2 * _NB :]

        sid = lax.axis_index("s")
        wid = sid * _NUM_CORES + lax.axis_index("c")
        base = wid * per_w
        chunk8 = (n_nodes // _NUM_SUBCORES) & ~7
        tail = n_nodes - chunk8 * _NUM_SUBCORES
        pltpu.sync_copy(
            x_hbm.at[pl.ds(sid * chunk8, chunk8)],
            xs.at[pl.ds(sid * chunk8, chunk8)],
        )
        if tail:

            @pl.when(sid == 0)
            def _():
                pltpu.sync_copy(
                    x_hbm.at[pl.ds(chunk8 * _NUM_SUBCORES, tail)],
                    xs.at[pl.ds(chunk8 * _NUM_SUBCORES, tail)],
                )
        pltpu.sync_copy(src_hbm.at[pl.ds(base, per_w)], idx_v)
        plsc.subcore_barrier()

        def gather_start(c, b):
            pltpu.async_copy(
                xs.at[idx_v.at[pl.ds(c * _CHUNK, _CHUNK)]], bufs[b], gsem[b]
            )

        def gather_wait(b):
            pltpu.make_async_copy(
                x_hbm.at[pl.ds(0, _CHUNK)], bufs[b], gsem[b]
            ).wait()

        def write_start(c, b):
            pltpu.async_copy(
                bufs[b], out_hbm.at[pl.ds(base + c * _CHUNK, _CHUNK)], wsem[b]
            )

        def write_wait(b):
            pltpu.make_async_copy(
                bufs[b], out_hbm.at[pl.ds(base, _CHUNK)], wsem[b]
            ).wait()

        for b in range(_NB):
            gather_start(b, b)

        def round_body(i, carry):
            c0 = i * _NB
            for b in range(_NB):
                gather_wait(b)
                write_start(c0 + b, b)
            for b in range(_NB):
                write_wait(b)
                nxt = c0 + _NB + b

                @pl.when(nxt < n_full)
                def _():
                    gather_start(nxt, b)

            return carry

        lax.fori_loop(0, n_rounds, round_body, 0, unroll=False)

        for b in range(n_full - n_rounds * _NB):
            gather_wait(b)
            write_start(n_rounds * _NB + b, b)
            write_wait(b)

        if tail_c:
            off = n_full * _CHUNK
            pltpu.async_copy(
                xs.at[idx_v.at[pl.ds(off, tail_c)]],
                bufs[0].at[pl.ds(0, tail_c)],
                gsem[0],
            ).wait()
            pltpu.sync_copy(
                bufs[0].at[pl.ds(0, tail_c)],
                out_hbm.at[pl.ds(base + off, tail_c)],
            )

    return body(x, src)


def kernel(x, edge_index):
    src = edge_index[0].astype(jnp.int32)
    return _sc_gather(x, src, n_edges=src.shape[0])


# speedup vs baseline: 1.0233x; 1.0233x over previous
import functools

import jax
import jax.numpy as jnp
from jax import lax
from jax.experimental import pallas as pl
from jax.experimental.pallas import tpu as pltpu
from jax.experimental.pallas import tpu_sc as plsc

_NUM_CORES = 2
_NUM_SUBCORES = 16
_NW = _NUM_CORES * _NUM_SUBCORES

_D = 128
_CHUNK = 40
_NB = 8


@functools.partial(jax.jit, static_argnames=("n_edges",))
def _sc_gather(x, src, *, n_edges):
    per_w = n_edges // _NW
    n_full = per_w // _CHUNK
    tail_c = per_w - n_full * _CHUNK
    n_rounds = n_full // _NB
    assert n_full >= _NB and tail_c % 8 == 0
    n_nodes = x.shape[0]
    rows_per_s = n_nodes // _NUM_SUBCORES
    mesh = plsc.VectorSubcoreMesh(core_axis_name="c", subcore_axis_name="s")

    scratch = (
        [
            pltpu.VMEM_SHARED((n_nodes, _D), jnp.float32),
            pltpu.VMEM((per_w,), jnp.int32),
        ]
        + [pltpu.VMEM((_CHUNK, _D), jnp.float32) for _ in range(_NB)]
        + [pltpu.SemaphoreType.DMA for _ in range(2 * _NB)]
    )

    @functools.partial(
        pl.kernel,
        out_type=jax.ShapeDtypeStruct((n_edges, _D), jnp.float32),
        mesh=mesh,
        scratch_types=scratch,
    )
    def body(x_hbm, src_hbm, out_hbm, xs, idx_v, *bufs_and_sems):
        bufs = bufs_and_sems[:_NB]
        gsem = bufs_and_sems[_NB : 2 * _NB]
        wsem = bufs_and_sems[2 * _NB :]

        sid = lax.axis_index("s")
        wid = sid * _NUM_CORES + lax.axis_index("c")
        base = wid * per_w
        chunk8 = (n_nodes // _NUM_SUBCORES) & ~7
        tail = n_nodes - chunk8 * _NUM_SUBCORES
        pltpu.sync_copy(
            x_hbm.at[pl.ds(sid * chunk8, chunk8)],
            xs.at[pl.ds(sid * chunk8, chunk8)],
        )
        if tail:

            @pl.when(sid == 0)
            def _():
                pltpu.sync_copy(
                    x_hbm.at[pl.ds(chunk8 * _NUM_SUBCORES, tail)],
                    xs.at[pl.ds(chunk8 * _NUM_SUBCORES, tail)],
                )
        pltpu.sync_copy(src_hbm.at[pl.ds(base, per_w)], idx_v)
        plsc.subcore_barrier()

        def gather_start(c, b):
            pltpu.async_copy(
                xs.at[idx_v.at[pl.ds(c * _CHUNK, _CHUNK)]], bufs[b], gsem[b]
            )

        def gather_wait(b):
            pltpu.make_async_copy(
                x_hbm.at[pl.ds(0, _CHUNK)], bufs[b], gsem[b]
            ).wait()

        def write_start(c, b):
            pltpu.async_copy(
                bufs[b], out_hbm.at[pl.ds(base + c * _CHUNK, _CHUNK)], wsem[b]
            )

        def write_wait(b):
            pltpu.make_async_copy(
                bufs[b], out_hbm.at[pl.ds(base, _CHUNK)], wsem[b]
            ).wait()

        for b in range(_NB):
            gather_start(b, b)

        def round_body(i, carry):
            c0 = i * _NB
            for b in range(_NB):
                gather_wait(b)
                write_start(c0 + b, b)
            for b in range(_NB):
                write_wait(b)
                nxt = c0 + _NB + b

                @pl.when(nxt < n_full)
                def _():
                    gather_start(nxt, b)

            return carry

        lax.fori_loop(0, n_rounds, round_body, 0, unroll=False)

        for b in range(n_full - n_rounds * _NB):
            gather_wait(b)
            write_start(n_rounds * _NB + b, b)
            write_wait(b)

        if tail_c:
            off = n_full * _CHUNK
            pltpu.async_copy(
                xs.at[idx_v.at[pl.ds(off, tail_c)]],
                bufs[0].at[pl.ds(0, tail_c)],
                gsem[0],
            ).wait()
            pltpu.sync_copy(
                bufs[0].at[pl.ds(0, tail_c)],
                out_hbm.at[pl.ds(base + off, tail_c)],
            )

    return body(x, src)


def kernel(x, edge_index):
    src = edge_index[0].astype(jnp.int32)
    return _sc_gather(x, src, n_edges=src.shape[0])


# speedup vs baseline: 1.0285x; 1.0050x over previous
import functools

import jax
import jax.numpy as jnp
from jax import lax
from jax.experimental import pallas as pl
from jax.experimental.pallas import tpu as pltpu
from jax.experimental.pallas import tpu_sc as plsc

_NUM_CORES = 2
_NUM_SUBCORES = 16
_NW = _NUM_CORES * _NUM_SUBCORES

_D = 128
_CHUNK = 40
_NB = 8


@functools.partial(jax.jit, static_argnames=("n_edges",))
def _sc_gather(x, src, *, n_edges):
    per_w = n_edges // _NW
    n_full = per_w // _CHUNK
    tail_c = per_w - n_full * _CHUNK
    n_rounds = n_full // _NB
    assert n_full >= _NB and tail_c % 8 == 0
    n_nodes = x.shape[0]
    rows_per_s = n_nodes // _NUM_SUBCORES
    mesh = plsc.VectorSubcoreMesh(core_axis_name="c", subcore_axis_name="s")

    scratch = (
        [
            pltpu.VMEM_SHARED((n_nodes, _D), jnp.float32),
            pltpu.VMEM((per_w,), jnp.int32),
        ]
        + [pltpu.VMEM((_CHUNK, _D), jnp.float32) for _ in range(_NB)]
        + [pltpu.SemaphoreType.DMA for _ in range(2 * _NB)]
    )

    @functools.partial(
        pl.kernel,
        out_type=jax.ShapeDtypeStruct((n_edges, _D), jnp.float32),
        mesh=mesh,
        scratch_types=scratch,
    )
    def body(x_hbm, src_hbm, out_hbm, xs, idx_v, *bufs_and_sems):
        bufs = bufs_and_sems[:_NB]
        gsem = bufs_and_sems[_NB : 2 * _NB]
        wsem = bufs_and_sems[2 * _NB :]

        sid = lax.axis_index("s")
        wid = sid * _NUM_CORES + lax.axis_index("c")
        base = wid * per_w
        pltpu.sync_copy(src_hbm.at[pl.ds(base, per_w)], idx_v)

        def gather_start(c, b, src_ref=None):
            src_ref = xs if src_ref is None else src_ref
            pltpu.async_copy(
                src_ref.at[idx_v.at[pl.ds(c * _CHUNK, _CHUNK)]], bufs[b], gsem[b]
            )

        for b in range(_NB):
            gather_start(b, b, x_hbm)

        chunk8 = (n_nodes // _NUM_SUBCORES) & ~7
        tail = n_nodes - chunk8 * _NUM_SUBCORES
        pltpu.sync_copy(
            x_hbm.at[pl.ds(sid * chunk8, chunk8)],
            xs.at[pl.ds(sid * chunk8, chunk8)],
        )
        if tail:

            @pl.when(sid == 0)
            def _():
                pltpu.sync_copy(
                    x_hbm.at[pl.ds(chunk8 * _NUM_SUBCORES, tail)],
                    xs.at[pl.ds(chunk8 * _NUM_SUBCORES, tail)],
                )
        plsc.subcore_barrier()

        def gather_wait(b):
            pltpu.make_async_copy(
                x_hbm.at[pl.ds(0, _CHUNK)], bufs[b], gsem[b]
            ).wait()

        def write_start(c, b):
            pltpu.async_copy(
                bufs[b], out_hbm.at[pl.ds(base + c * _CHUNK, _CHUNK)], wsem[b]
            )

        def write_wait(b):
            pltpu.make_async_copy(
                bufs[b], out_hbm.at[pl.ds(base, _CHUNK)], wsem[b]
            ).wait()

        def round_body(i, carry):
            c0 = i * _NB
            for b in range(_NB):
                gather_wait(b)
                write_start(c0 + b, b)
            for b in range(_NB):
                write_wait(b)
                nxt = c0 + _NB + b

                @pl.when(nxt < n_full)
                def _():
                    gather_start(nxt, b)

            return carry

        lax.fori_loop(0, n_rounds, round_body, 0, unroll=False)

        for b in range(n_full - n_rounds * _NB):
            gather_wait(b)
            write_start(n_rounds * _NB + b, b)
            write_wait(b)

        if tail_c:
            off = n_full * _CHUNK
            pltpu.async_copy(
                xs.at[idx_v.at[pl.ds(off, tail_c)]],
                bufs[0].at[pl.ds(0, tail_c)],
                gsem[0],
            ).wait()
            pltpu.sync_copy(
                bufs[0].at[pl.ds(0, tail_c)],
                out_hbm.at[pl.ds(base + off, tail_c)],
            )

    return body(x, src)


def kernel(x, edge_index):
    src = edge_index[0].astype(jnp.int32)
    return _sc_gather(x, src, n_edges=src.shape[0])
